# baseline (speedup 1.0000x reference)
import functools

import jax
import jax.numpy as jnp
from jax import lax
from jax.experimental import pallas as pl
from jax.experimental.pallas import tpu as pltpu
from jax.experimental.pallas import tpu_sc as plsc

N = 10000
E = 320000
EL = 20000

NP = 10240
EC = 2560
EP = EC * 128
CPS = EC // 16
ELC = 160
ELP = ELC * 128
HF = 64

_MESH = plsc.VectorSubcoreMesh(core_axis_name="c", subcore_axis_name="s")
_SC_PARAMS = pltpu.CompilerParams(needs_layout_passes=False, use_tc_tiling_on_sc=False)

_Z16F = functools.partial(jnp.zeros, (16,), jnp.float32)


def _rsqrt16(x):
    i = plsc.bitcast(x, jnp.int32)
    i = jnp.int32(0x5F3759DF) - (i >> 1)
    y = plsc.bitcast(i, jnp.float32)
    for _ in range(3):
        y = y * (1.5 - 0.5 * x * y * y)
    return y


def _dis_body(col_hbm, ew_hbm, dis_hbm, col_v, ew_v, deg_v, st_v, iota_v, deg_sh):
    s = lax.axis_index("s")
    c = lax.axis_index("c")
    zero16 = _Z16F()
    lanes = lax.iota(jnp.int32, 16)

    pltpu.sync_copy(col_hbm.at[s], col_v)
    pltpu.sync_copy(ew_hbm.at[s], ew_v)

    for k in range(5):
        iota_v[pl.ds(k * 16, 16)] = lanes + k * 16

    @pl.when(s < 10)
    def _():
        for r in range(8):
            for f in range(8):
                st_v[r, pl.ds(f * 16, 16)] = zero16
        pltpu.sync_copy(st_v, deg_sh.at[pl.ds(s * 8, 8)])

    def _zb(r, _):
        for f in range(8):
            deg_v[r, pl.ds(f * 16, 16)] = zero16
        return 0

    lax.fori_loop(0, 80, _zb, 0)

    def _eb(j, _):
        for v in range(4):
            lo, hi = plsc.unpack(col_v[j, pl.ds(v * 32, 32)],
                                 format=plsc.PackFormat.INTERLEAVED,
                                 preferred_element_type=jnp.int32)
            w0 = ew_v[j, pl.ds(v * 32, 16)]
            w1 = ew_v[j, pl.ds(v * 32 + 16, 16)]
            plsc.addupdate_scatter(deg_v, [lo >> 7, lo & 127], w0)
            plsc.addupdate_scatter(deg_v, [hi >> 7, hi & 127], w1)
        return 0

    lax.fori_loop(0, CPS, _eb, 0)

    plsc.subcore_barrier()
    pltpu.sync_copy(deg_v, deg_sh.at[iota_v], add=True)
    plsc.subcore_barrier()

    @pl.when(jnp.logical_and(s < 10, c == 0))
    def _():
        pltpu.sync_copy(deg_sh.at[pl.ds(s * 8, 8)], st_v)
        for r in range(8):
            for f in range(8):
                d = st_v[r, pl.ds(f * 16, 16)] + 1.0
                st_v[r, pl.ds(f * 16, 16)] = _rsqrt16(d)
        pltpu.sync_copy(st_v, dis_hbm.at[pl.ds(s * 8, 8)])


_dis_call = pl.kernel(
    _dis_body,
    out_type=jax.ShapeDtypeStruct((80, 128), jnp.float32),
    mesh=_MESH,
    compiler_params=_SC_PARAMS,
    scratch_types=[
        pltpu.VMEM((CPS, 128), jnp.int16),
        pltpu.VMEM((CPS, 128), jnp.float32),
        pltpu.VMEM((80, 128), jnp.float32),
        pltpu.VMEM((8, 128), jnp.float32),
        pltpu.VMEM((80,), jnp.int32),
        pltpu.VMEM_SHARED((80, 128), jnp.float32),
    ],
)


def _msg_body(g_hbm, row_hbm, col_hbm, ew_hbm, acc_hbm,
              row_v, col_v, ew_v, buf, z_v, riw, ciw, acc_sh, gsem, ssem):
    s = lax.axis_index("s")
    c = lax.axis_index("c")
    zero16 = _Z16F()

    pltpu.sync_copy(row_hbm.at[s], row_v)
    pltpu.sync_copy(col_hbm.at[s], col_v)
    pltpu.sync_copy(ew_hbm.at[s], ew_v)

    def _cvt(src_v, j, dst, slot):
        for k in range(4):
            lo, hi = plsc.unpack(src_v[j, pl.ds(k * 32, 32)],
                                 format=plsc.PackFormat.INTERLEAVED,
                                 preferred_element_type=jnp.int32)
            dst[slot, pl.ds(k * 32, 16)] = lo
            dst[slot, pl.ds(k * 32 + 16, 16)] = hi

    def _zb(r, _):
        for f in range(4):
            z_v[r, pl.ds(f * 16, 16)] = zero16
        return 0

    lax.fori_loop(0, 128, _zb, 0)
    for k in range(5):
        pltpu.sync_copy(z_v, acc_sh.at[pl.ds(s * 640 + k * 128, 128)])
    plsc.subcore_barrier()

    _cvt(row_v, 0, riw, 0)
    pltpu.async_copy(g_hbm.at[c].at[riw.at[0]], buf.at[0], gsem.at[0])

    def _chunk(j, _):
        b = lax.rem(j, 3)
        bn = lax.rem(j + 1, 3)
        pltpu.make_async_copy(
            acc_hbm.at[0].at[pl.ds(0, 128)], buf.at[b], gsem.at[b]).wait()

        @pl.when(j >= 2)
        def _():
            pltpu.make_async_copy(
                buf.at[bn], acc_sh.at[ciw.at[0]], ssem.at[bn]).wait()

        @pl.when(j < CPS - 1)
        def _():
            _cvt(row_v, j + 1, riw, bn)
            pltpu.async_copy(g_hbm.at[c].at[riw.at[bn]],
                             buf.at[bn], gsem.at[bn])

        row_w = ew_v.at[j]

        def _e4(i, __):
            for u in range(4):
                e = i * 4 + u
                w = plsc.load_gather(row_w, [jnp.zeros((16,), jnp.int32) + e])
                for f in range(4):
                    buf[b, e, pl.ds(f * 16, 16)] = \
                        buf[b, e, pl.ds(f * 16, 16)] * w
            return 0

        lax.fori_loop(0, 32, _e4, 0)
        _cvt(col_v, j, ciw, b)
        pltpu.async_copy(buf.at[b], acc_sh.at[ciw.at[b]], ssem.at[b],
                         add=True)
        return 0

    lax.fori_loop(0, CPS, _chunk, 0)
    for jt in (CPS - 2, CPS - 1):
        pltpu.make_async_copy(
            buf.at[jt % 3], acc_sh.at[ciw.at[0]], ssem.at[jt % 3]).wait()
    plsc.subcore_barrier()
    pltpu.sync_copy(acc_sh.at[pl.ds(s * 640, 640)],
                    acc_hbm.at[c].at[pl.ds(s * 640, 640)])


_msg_call = pl.kernel(
    _msg_body,
    out_type=jax.ShapeDtypeStruct((2, NP, HF), jnp.float32),
    mesh=_MESH,
    compiler_params=_SC_PARAMS,
    scratch_types=[
        pltpu.VMEM((CPS, 128), jnp.int16),
        pltpu.VMEM((CPS, 128), jnp.int16),
        pltpu.VMEM((CPS, 128), jnp.float32),
        pltpu.VMEM((3, 128, HF), jnp.float32),
        pltpu.VMEM((128, HF), jnp.float32),
        pltpu.VMEM((3, 128), jnp.int32),
        pltpu.VMEM((3, 128), jnp.int32),
        pltpu.VMEM_SHARED((NP, HF), jnp.float32),
        pltpu.SemaphoreType.DMA((3,)),
        pltpu.SemaphoreType.DMA((3,)),
    ],
)


def _dec_body(z_hbm, l_hbm, r_hbm, out_hbm,
              l_v, r_v, bufl, bufr, pacc, out_v, seml, semr):
    s = lax.axis_index("s")
    c = lax.axis_index("c")
    wid = c * 16 + s
    lanes = lax.iota(jnp.int32, 16)

    pltpu.sync_copy(l_hbm.at[wid], l_v)
    pltpu.sync_copy(r_hbm.at[wid], r_v)

    def _chunk(j, _):
        cl = pltpu.async_copy(z_hbm.at[l_v.at[j]], bufl, seml)
        cr = pltpu.async_copy(z_hbm.at[r_v.at[j]], bufr, semr)
        cl.wait()
        cr.wait()

        def _pair(e, __):
            acc = _Z16F()
            for f in range(8):
                acc = acc + bufl[e, pl.ds(f * 16, 16)] * bufr[e, pl.ds(f * 16, 16)]
            pacc[e, pl.ds(0, 16)] = acc
            return 0

        lax.fori_loop(0, 128, _pair, 0)

        for v in range(8):
            idxp = lanes + v * 16
            tot = _Z16F()
            for l in range(16):
                tot = tot + plsc.load_gather(
                    pacc, [idxp, jnp.full((16,), l, jnp.int32)])
            out_v[j, pl.ds(v * 16, 16)] = tot
        return 0

    lax.fori_loop(0, 5, _chunk, 0)
    pltpu.sync_copy(out_v, out_hbm.at[wid])


_dec_call = pl.kernel(
    _dec_body,
    out_type=jax.ShapeDtypeStruct((32, 5, 128), jnp.float32),
    mesh=_MESH,
    compiler_params=_SC_PARAMS,
    scratch_types=[
        pltpu.VMEM((5, 128), jnp.int32),
        pltpu.VMEM((5, 128), jnp.int32),
        pltpu.VMEM((128, 128), jnp.float32),
        pltpu.VMEM((128, 128), jnp.float32),
        pltpu.VMEM((128, 16), jnp.float32),
        pltpu.VMEM((5, 128), jnp.float32),
        pltpu.SemaphoreType.DMA,
        pltpu.SemaphoreType.DMA,
    ],
)


_RB = 256
_NBLK = NP // _RB


def _mm_scale_body(x_ref, w_ref, dis_ref, o_ref):
    g = jnp.dot(x_ref[...], w_ref[...],
                preferred_element_type=jnp.float32) * dis_ref[...]
    o_ref[0] = g[:, :HF]
    o_ref[1] = g[:, HF:]


def _tc_mm_scale(x, w, dis_col):
    return pl.pallas_call(
        _mm_scale_body,
        grid=(_NBLK,),
        in_specs=[
            pl.BlockSpec((_RB, 128), lambda i: (i, 0)),
            pl.BlockSpec((128, 128), lambda i: (0, 0)),
            pl.BlockSpec((_RB, 1), lambda i: (i, 0)),
        ],
        out_specs=pl.BlockSpec((2, _RB, HF), lambda i: (0, i, 0)),
        out_shape=jax.ShapeDtypeStruct((2, NP, HF), jnp.float32),
    )(x, w, dis_col)


def _layer2_body(a_ref, g_ref, dis_ref, b_ref, w_ref, o_ref):
    pre = jnp.concatenate(
        [a_ref[0] + g_ref[0], a_ref[1] + g_ref[1]], axis=-1)
    h = jnp.maximum(dis_ref[...] * pre + b_ref[...], 0.0)
    g2 = jnp.dot(h, w_ref[...],
                 preferred_element_type=jnp.float32) * dis_ref[...]
    o_ref[0] = g2[:, :HF]
    o_ref[1] = g2[:, HF:]


def _tc_layer2(acc, g1, dis_col, b1, w2):
    return pl.pallas_call(
        _layer2_body,
        grid=(_NBLK,),
        in_specs=[
            pl.BlockSpec((2, _RB, HF), lambda i: (0, i, 0)),
            pl.BlockSpec((2, _RB, HF), lambda i: (0, i, 0)),
            pl.BlockSpec((_RB, 1), lambda i: (i, 0)),
            pl.BlockSpec((1, 128), lambda i: (0, 0)),
            pl.BlockSpec((128, 128), lambda i: (0, 0)),
        ],
        out_specs=pl.BlockSpec((2, _RB, HF), lambda i: (0, i, 0)),
        out_shape=jax.ShapeDtypeStruct((2, NP, HF), jnp.float32),
    )(acc, g1, dis_col, b1, w2)


def _final_body(a_ref, g_ref, dis_ref, b_ref, o_ref):
    pre = jnp.concatenate(
        [a_ref[0] + g_ref[0], a_ref[1] + g_ref[1]], axis=-1)
    o_ref[...] = dis_ref[...] * pre + b_ref[...]


def _tc_final(acc, g2, dis_col, b2):
    return pl.pallas_call(
        _final_body,
        grid=(_NBLK,),
        in_specs=[
            pl.BlockSpec((2, _RB, HF), lambda i: (0, i, 0)),
            pl.BlockSpec((2, _RB, HF), lambda i: (0, i, 0)),
            pl.BlockSpec((_RB, 1), lambda i: (i, 0)),
            pl.BlockSpec((1, 128), lambda i: (0, 0)),
        ],
        out_specs=pl.BlockSpec((_RB, 128), lambda i: (i, 0)),
        out_shape=jax.ShapeDtypeStruct((NP, 128), jnp.float32),
    )(acc, g2, dis_col, b2)


def _pad_to(a, total, fill):
    return jnp.concatenate([a, jnp.full((total - a.shape[0],), fill, a.dtype)])


def _pack_idx(a):
    p = _pad_to(a, EP, 0).astype(jnp.int16)
    return p.reshape(-1, 2, 16).transpose(0, 2, 1).reshape(16, CPS, 128)


def kernel(x, edge_index, edge_weight, edge_label_index, W1, b1, W2, b2):
    row16 = _pack_idx(edge_index[0])
    col16 = _pack_idx(edge_index[1])
    ew16 = _pad_to(edge_weight, EP, 0.0).reshape(16, CPS, 128)
    x_p = jnp.pad(x, ((0, NP - N), (0, 0)))

    dis2d = _dis_call(col16, ew16)
    dis_col = dis2d.reshape(NP, 1)

    g1 = _tc_mm_scale(x_p, W1, dis_col)
    acc1 = _msg_call(g1, row16, col16, ew16)
    g2 = _tc_layer2(acc1, g1, dis_col, b1.reshape(1, 128), W2)
    acc2 = _msg_call(g2, row16, col16, ew16)
    z = _tc_final(acc2, g2, dis_col, b2.reshape(1, 128))

    lp = _pad_to(edge_label_index[0], ELP, 0).reshape(32, 5, 128)
    rp = _pad_to(edge_label_index[1], ELP, 0).reshape(32, 5, 128)
    logits = _dec_call(z, lp, rp)
    return logits.reshape(-1)[:EL]


# speedup vs baseline: 1.2185x; 1.2185x over previous
import functools

import jax
import jax.numpy as jnp
from jax import lax
from jax.experimental import pallas as pl
from jax.experimental.pallas import tpu as pltpu
from jax.experimental.pallas import tpu_sc as plsc

N = 10000
E = 320000
EL = 20000

NP = 10240
EC = 2560
EP = EC * 128
CPS = EC // 16
ELC = 160
ELP = ELC * 128
HF = 64

_MESH = plsc.VectorSubcoreMesh(core_axis_name="c", subcore_axis_name="s")
_SC_PARAMS = pltpu.CompilerParams(needs_layout_passes=False, use_tc_tiling_on_sc=False)

_Z16F = functools.partial(jnp.zeros, (16,), jnp.float32)


def _rsqrt16(x):
    i = plsc.bitcast(x, jnp.int32)
    i = jnp.int32(0x5F3759DF) - (i >> 1)
    y = plsc.bitcast(i, jnp.float32)
    for _ in range(3):
        y = y * (1.5 - 0.5 * x * y * y)
    return y


def _dis_body(col_hbm, ew_hbm, dis_hbm, col_v, ew_v, deg_v, st_v, iota_v, deg_sh):
    s = lax.axis_index("s")
    c = lax.axis_index("c")
    zero16 = _Z16F()
    lanes = lax.iota(jnp.int32, 16)

    pltpu.sync_copy(col_hbm.at[s], col_v)
    pltpu.sync_copy(ew_hbm.at[s], ew_v)

    for k in range(5):
        iota_v[pl.ds(k * 16, 16)] = lanes + k * 16

    @pl.when(s < 10)
    def _():
        for r in range(8):
            for f in range(8):
                st_v[r, pl.ds(f * 16, 16)] = zero16
        pltpu.sync_copy(st_v, deg_sh.at[pl.ds(s * 8, 8)])

    def _zb(r, _):
        for f in range(8):
            deg_v[r, pl.ds(f * 16, 16)] = zero16
        return 0

    lax.fori_loop(0, 80, _zb, 0)

    def _eb(j, _):
        for v in range(4):
            lo, hi = plsc.unpack(col_v[j, pl.ds(v * 32, 32)],
                                 format=plsc.PackFormat.INTERLEAVED,
                                 preferred_element_type=jnp.int32)
            w0 = ew_v[j, pl.ds(v * 32, 16)]
            w1 = ew_v[j, pl.ds(v * 32 + 16, 16)]
            plsc.addupdate_scatter(deg_v, [lo >> 7, lo & 127], w0)
            plsc.addupdate_scatter(deg_v, [hi >> 7, hi & 127], w1)
        return 0

    lax.fori_loop(0, CPS, _eb, 0)

    plsc.subcore_barrier()
    pltpu.sync_copy(deg_v, deg_sh.at[iota_v], add=True)
    plsc.subcore_barrier()

    @pl.when(jnp.logical_and(s < 10, c == 0))
    def _():
        pltpu.sync_copy(deg_sh.at[pl.ds(s * 8, 8)], st_v)
        for r in range(8):
            for f in range(8):
                d = st_v[r, pl.ds(f * 16, 16)] + 1.0
                st_v[r, pl.ds(f * 16, 16)] = _rsqrt16(d)
        pltpu.sync_copy(st_v, dis_hbm.at[pl.ds(s * 8, 8)])


_dis_call = pl.kernel(
    _dis_body,
    out_type=jax.ShapeDtypeStruct((80, 128), jnp.float32),
    mesh=_MESH,
    compiler_params=_SC_PARAMS,
    scratch_types=[
        pltpu.VMEM((CPS, 128), jnp.int16),
        pltpu.VMEM((CPS, 128), jnp.float32),
        pltpu.VMEM((80, 128), jnp.float32),
        pltpu.VMEM((8, 128), jnp.float32),
        pltpu.VMEM((80,), jnp.int32),
        pltpu.VMEM_SHARED((80, 128), jnp.float32),
    ],
)


def _msg_body(g_hbm, row_hbm, col_hbm, ew_hbm, acc_hbm,
              row_v, col_v, ew_v, buf, z_v, riw, ciw, acc_sh, gsem):
    s = lax.axis_index("s")
    c = lax.axis_index("c")
    zero16 = _Z16F()

    pltpu.sync_copy(row_hbm.at[s], row_v)
    pltpu.sync_copy(col_hbm.at[s], col_v)
    pltpu.sync_copy(ew_hbm.at[s], ew_v)

    def _cvt(src_v, j, dst, slot):
        for k in range(4):
            lo, hi = plsc.unpack(src_v[j, pl.ds(k * 32, 32)],
                                 format=plsc.PackFormat.INTERLEAVED,
                                 preferred_element_type=jnp.int32)
            dst[slot, pl.ds(k * 32, 16)] = lo
            dst[slot, pl.ds(k * 32 + 16, 16)] = hi

    def _zb(r, _):
        for f in range(4):
            z_v[r, pl.ds(f * 16, 16)] = zero16
        return 0

    lax.fori_loop(0, 128, _zb, 0)
    for k in range(5):
        pltpu.sync_copy(z_v, acc_sh.at[pl.ds(s * 640 + k * 128, 128)])
    plsc.subcore_barrier()

    _cvt(row_v, 0, riw, 0)
    pltpu.async_copy(g_hbm.at[c].at[riw.at[0]], buf.at[0], gsem.at[0])

    def _chunk(j, _):
        b = j & 1
        pltpu.make_async_copy(
            acc_hbm.at[0].at[pl.ds(0, 128)], buf.at[b], gsem.at[b]).wait()

        @pl.when(j < CPS - 1)
        def _():
            _cvt(row_v, j + 1, riw, 1 - b)
            pltpu.async_copy(g_hbm.at[c].at[riw.at[1 - b]],
                             buf.at[1 - b], gsem.at[1 - b])

        row_w = ew_v.at[j]

        def _e4(i, __):
            for u in range(4):
                e = i * 4 + u
                w = plsc.load_gather(row_w, [jnp.zeros((16,), jnp.int32) + e])
                for f in range(4):
                    buf[b, e, pl.ds(f * 16, 16)] = \
                        buf[b, e, pl.ds(f * 16, 16)] * w
            return 0

        lax.fori_loop(0, 32, _e4, 0)
        _cvt(col_v, j, ciw, b)
        pltpu.sync_copy(buf.at[b], acc_sh.at[ciw.at[b]], add=True)
        return 0

    lax.fori_loop(0, CPS, _chunk, 0)
    plsc.subcore_barrier()
    pltpu.sync_copy(acc_sh.at[pl.ds(s * 640, 640)],
                    acc_hbm.at[c].at[pl.ds(s * 640, 640)])


_msg_call = pl.kernel(
    _msg_body,
    out_type=jax.ShapeDtypeStruct((2, NP, HF), jnp.float32),
    mesh=_MESH,
    compiler_params=_SC_PARAMS,
    scratch_types=[
        pltpu.VMEM((CPS, 128), jnp.int16),
        pltpu.VMEM((CPS, 128), jnp.int16),
        pltpu.VMEM((CPS, 128), jnp.float32),
        pltpu.VMEM((2, 128, HF), jnp.float32),
        pltpu.VMEM((128, HF), jnp.float32),
        pltpu.VMEM((2, 128), jnp.int32),
        pltpu.VMEM((2, 128), jnp.int32),
        pltpu.VMEM_SHARED((NP, HF), jnp.float32),
        pltpu.SemaphoreType.DMA((2,)),
    ],
)


def _dec_body(z_hbm, l_hbm, r_hbm, out_hbm,
              l_v, r_v, bufl, bufr, pacc, out_v, seml, semr):
    s = lax.axis_index("s")
    c = lax.axis_index("c")
    wid = c * 16 + s
    lanes = lax.iota(jnp.int32, 16)

    pltpu.sync_copy(l_hbm.at[wid], l_v)
    pltpu.sync_copy(r_hbm.at[wid], r_v)

    def _chunk(j, _):
        cl = pltpu.async_copy(z_hbm.at[l_v.at[j]], bufl, seml)
        cr = pltpu.async_copy(z_hbm.at[r_v.at[j]], bufr, semr)
        cl.wait()
        cr.wait()

        def _pair(e, __):
            acc = _Z16F()
            for f in range(8):
                acc = acc + bufl[e, pl.ds(f * 16, 16)] * bufr[e, pl.ds(f * 16, 16)]
            pacc[e, pl.ds(0, 16)] = acc
            return 0

        lax.fori_loop(0, 128, _pair, 0)

        for v in range(8):
            idxp = lanes + v * 16
            tot = _Z16F()
            for l in range(16):
                tot = tot + plsc.load_gather(
                    pacc, [idxp, jnp.full((16,), l, jnp.int32)])
            out_v[j, pl.ds(v * 16, 16)] = tot
        return 0

    lax.fori_loop(0, 5, _chunk, 0)
    pltpu.sync_copy(out_v, out_hbm.at[wid])


_dec_call = pl.kernel(
    _dec_body,
    out_type=jax.ShapeDtypeStruct((32, 5, 128), jnp.float32),
    mesh=_MESH,
    compiler_params=_SC_PARAMS,
    scratch_types=[
        pltpu.VMEM((5, 128), jnp.int32),
        pltpu.VMEM((5, 128), jnp.int32),
        pltpu.VMEM((128, 128), jnp.float32),
        pltpu.VMEM((128, 128), jnp.float32),
        pltpu.VMEM((128, 16), jnp.float32),
        pltpu.VMEM((5, 128), jnp.float32),
        pltpu.SemaphoreType.DMA,
        pltpu.SemaphoreType.DMA,
    ],
)


_RB = 256
_NBLK = NP // _RB


def _mm_scale_body(x_ref, w_ref, dis_ref, o_ref):
    g = jnp.dot(x_ref[...], w_ref[...],
                preferred_element_type=jnp.float32) * dis_ref[...]
    o_ref[0] = g[:, :HF]
    o_ref[1] = g[:, HF:]


def _tc_mm_scale(x, w, dis_col):
    return pl.pallas_call(
        _mm_scale_body,
        grid=(_NBLK,),
        in_specs=[
            pl.BlockSpec((_RB, 128), lambda i: (i, 0)),
            pl.BlockSpec((128, 128), lambda i: (0, 0)),
            pl.BlockSpec((_RB, 1), lambda i: (i, 0)),
        ],
        out_specs=pl.BlockSpec((2, _RB, HF), lambda i: (0, i, 0)),
        out_shape=jax.ShapeDtypeStruct((2, NP, HF), jnp.float32),
    )(x, w, dis_col)


def _layer2_body(a_ref, g_ref, dis_ref, b_ref, w_ref, o_ref):
    pre = jnp.concatenate(
        [a_ref[0] + g_ref[0], a_ref[1] + g_ref[1]], axis=-1)
    h = jnp.maximum(dis_ref[...] * pre + b_ref[...], 0.0)
    g2 = jnp.dot(h, w_ref[...],
                 preferred_element_type=jnp.float32) * dis_ref[...]
    o_ref[0] = g2[:, :HF]
    o_ref[1] = g2[:, HF:]


def _tc_layer2(acc, g1, dis_col, b1, w2):
    return pl.pallas_call(
        _layer2_body,
        grid=(_NBLK,),
        in_specs=[
            pl.BlockSpec((2, _RB, HF), lambda i: (0, i, 0)),
            pl.BlockSpec((2, _RB, HF), lambda i: (0, i, 0)),
            pl.BlockSpec((_RB, 1), lambda i: (i, 0)),
            pl.BlockSpec((1, 128), lambda i: (0, 0)),
            pl.BlockSpec((128, 128), lambda i: (0, 0)),
        ],
        out_specs=pl.BlockSpec((2, _RB, HF), lambda i: (0, i, 0)),
        out_shape=jax.ShapeDtypeStruct((2, NP, HF), jnp.float32),
    )(acc, g1, dis_col, b1, w2)


def _final_body(a_ref, g_ref, dis_ref, b_ref, o_ref):
    pre = jnp.concatenate(
        [a_ref[0] + g_ref[0], a_ref[1] + g_ref[1]], axis=-1)
    o_ref[...] = dis_ref[...] * pre + b_ref[...]


def _tc_final(acc, g2, dis_col, b2):
    return pl.pallas_call(
        _final_body,
        grid=(_NBLK,),
        in_specs=[
            pl.BlockSpec((2, _RB, HF), lambda i: (0, i, 0)),
            pl.BlockSpec((2, _RB, HF), lambda i: (0, i, 0)),
            pl.BlockSpec((_RB, 1), lambda i: (i, 0)),
            pl.BlockSpec((1, 128), lambda i: (0, 0)),
        ],
        out_specs=pl.BlockSpec((_RB, 128), lambda i: (i, 0)),
        out_shape=jax.ShapeDtypeStruct((NP, 128), jnp.float32),
    )(acc, g2, dis_col, b2)


def _pad_to(a, total, fill):
    return jnp.concatenate([a, jnp.full((total - a.shape[0],), fill, a.dtype)])


def _pack_idx(a):
    p = _pad_to(a, EP, 0).astype(jnp.int16)
    return p.reshape(-1, 2, 16).transpose(0, 2, 1).reshape(16, CPS, 128)


def kernel(x, edge_index, edge_weight, edge_label_index, W1, b1, W2, b2):
    row16 = _pack_idx(edge_index[0])
    col16 = _pack_idx(edge_index[1])
    ew16 = _pad_to(edge_weight, EP, 0.0).reshape(16, CPS, 128)
    x_p = jnp.pad(x, ((0, NP - N), (0, 0)))

    dis2d = _dis_call(col16, ew16)
    dis_col = dis2d.reshape(NP, 1)

    g1 = _tc_mm_scale(x_p, W1, dis_col)
    acc1 = _msg_call(g1, row16, col16, ew16)
    g2 = _tc_layer2(acc1, g1, dis_col, b1.reshape(1, 128), W2)
    acc2 = _msg_call(g2, row16, col16, ew16)
    z = _tc_final(acc2, g2, dis_col, b2.reshape(1, 128))

    lp = _pad_to(edge_label_index[0], ELP, 0).reshape(32, 5, 128)
    rp = _pad_to(edge_label_index[1], ELP, 0).reshape(32, 5, 128)
    logits = _dec_call(z, lp, rp)
    return logits.reshape(-1)[:EL]


# speedup vs baseline: 1.5618x; 1.2818x over previous
import functools

import jax
import jax.numpy as jnp
from jax import lax
from jax.experimental import pallas as pl
from jax.experimental.pallas import tpu as pltpu
from jax.experimental.pallas import tpu_sc as plsc

N = 10000
E = 320000
EL = 20000

NP = 10240
EC = 2560
EP = EC * 128
CPS = EC // 16
ELC = 160
ELP = ELC * 128
HF = 64

_MESH = plsc.VectorSubcoreMesh(core_axis_name="c", subcore_axis_name="s")
_SC_PARAMS = pltpu.CompilerParams(needs_layout_passes=False,
                                  use_tc_tiling_on_sc=False)

_Z16F = functools.partial(jnp.zeros, (16,), jnp.float32)


def _rsqrt16(x):
    i = plsc.bitcast(x, jnp.int32)
    i = jnp.int32(0x5F3759DF) - (i >> 1)
    y = plsc.bitcast(i, jnp.float32)
    for _ in range(3):
        y = y * (1.5 - 0.5 * x * y * y)
    return y


def _dis_body(col_hbm, ew_hbm, dis_hbm, col_v, ew_v, deg_v, st_v, iota_v, deg_sh):
    s = lax.axis_index("s")
    c = lax.axis_index("c")
    zero16 = _Z16F()
    lanes = lax.iota(jnp.int32, 16)

    pltpu.sync_copy(col_hbm.at[s], col_v)
    pltpu.sync_copy(ew_hbm.at[s], ew_v)

    for k in range(5):
        iota_v[pl.ds(k * 16, 16)] = lanes + k * 16

    @pl.when(s < 10)
    def _():
        for r in range(8):
            for f in range(8):
                st_v[r, pl.ds(f * 16, 16)] = zero16
        pltpu.sync_copy(st_v, deg_sh.at[pl.ds(s * 8, 8)])

    def _zb(r, _):
        for f in range(8):
            deg_v[r, pl.ds(f * 16, 16)] = zero16
        return 0

    lax.fori_loop(0, 80, _zb, 0)

    def _eb(j, _):
        for v in range(8):
            idx = col_v[j, pl.ds(v * 16, 16)]
            w = ew_v[j, pl.ds(v * 16, 16)]
            plsc.addupdate_scatter(deg_v, [idx >> 7, idx & 127], w)
        return 0

    lax.fori_loop(0, CPS, _eb, 0)

    plsc.subcore_barrier()
    pltpu.sync_copy(deg_v, deg_sh.at[iota_v], add=True)
    plsc.subcore_barrier()

    @pl.when(jnp.logical_and(s < 10, c == 0))
    def _():
        pltpu.sync_copy(deg_sh.at[pl.ds(s * 8, 8)], st_v)
        for r in range(8):
            for f in range(8):
                d = st_v[r, pl.ds(f * 16, 16)] + 1.0
                st_v[r, pl.ds(f * 16, 16)] = _rsqrt16(d)
        pltpu.sync_copy(st_v, dis_hbm.at[pl.ds(s * 8, 8)])


_dis_call = pl.kernel(
    _dis_body,
    out_type=jax.ShapeDtypeStruct((80, 128), jnp.float32),
    mesh=_MESH,
    compiler_params=_SC_PARAMS,
    scratch_types=[
        pltpu.VMEM((CPS, 128), jnp.int32),
        pltpu.VMEM((CPS, 128), jnp.float32),
        pltpu.VMEM((80, 128), jnp.float32),
        pltpu.VMEM((8, 128), jnp.float32),
        pltpu.VMEM((80,), jnp.int32),
        pltpu.VMEM_SHARED((80, 128), jnp.float32),
    ],
)


def _msg_body(g_hbm, row_hbm, col_hbm, ew_hbm, acc_hbm,
              row_v, col_v, ew_v, buf, z_v, acc_sh, gsem):
    s = lax.axis_index("s")
    c = lax.axis_index("c")
    zero16 = _Z16F()

    pltpu.sync_copy(row_hbm.at[s], row_v)
    pltpu.sync_copy(col_hbm.at[s], col_v)
    pltpu.sync_copy(ew_hbm.at[s], ew_v)

    def _zb(r, _):
        for f in range(4):
            z_v[r, pl.ds(f * 16, 16)] = zero16
        return 0

    lax.fori_loop(0, 128, _zb, 0)
    for k in range(5):
        pltpu.sync_copy(z_v, acc_sh.at[pl.ds(s * 640 + k * 128, 128)])
    plsc.subcore_barrier()

    pltpu.async_copy(g_hbm.at[c].at[row_v.at[0]], buf.at[0], gsem.at[0])

    def _chunk(j, _):
        b = j & 1
        pltpu.make_async_copy(
            acc_hbm.at[0].at[pl.ds(0, 128)], buf.at[b], gsem.at[b]).wait()

        @pl.when(j < CPS - 1)
        def _():
            pltpu.async_copy(g_hbm.at[c].at[row_v.at[j + 1]],
                             buf.at[1 - b], gsem.at[1 - b])

        row_w = ew_v.at[j]

        def _e8(i, __):
            for u in range(8):
                e = i * 8 + u
                w = plsc.load_gather(row_w, [jnp.zeros((16,), jnp.int32) + e])
                for f in range(4):
                    buf[b, e, pl.ds(f * 16, 16)] = \
                        buf[b, e, pl.ds(f * 16, 16)] * w
            return 0

        lax.fori_loop(0, 16, _e8, 0)
        pltpu.sync_copy(buf.at[b], acc_sh.at[col_v.at[j]], add=True)
        return 0

    lax.fori_loop(0, CPS, _chunk, 0)
    plsc.subcore_barrier()
    pltpu.sync_copy(acc_sh.at[pl.ds(s * 640, 640)],
                    acc_hbm.at[c].at[pl.ds(s * 640, 640)])


_msg_call = pl.kernel(
    _msg_body,
    out_type=jax.ShapeDtypeStruct((2, NP, HF), jnp.float32),
    mesh=_MESH,
    compiler_params=_SC_PARAMS,
    scratch_types=[
        pltpu.VMEM((CPS, 128), jnp.int32),
        pltpu.VMEM((CPS, 128), jnp.int32),
        pltpu.VMEM((CPS, 128), jnp.float32),
        pltpu.VMEM((2, 128, HF), jnp.float32),
        pltpu.VMEM((128, HF), jnp.float32),
        pltpu.VMEM_SHARED((NP, HF), jnp.float32),
        pltpu.SemaphoreType.DMA((2,)),
    ],
)


def _dec_body(z_hbm, l_hbm, r_hbm, out_hbm,
              l_v, r_v, bufl, bufr, pacc, out_v, seml, semr):
    s = lax.axis_index("s")
    c = lax.axis_index("c")
    wid = c * 16 + s
    lanes = lax.iota(jnp.int32, 16)

    pltpu.sync_copy(l_hbm.at[wid], l_v)
    pltpu.sync_copy(r_hbm.at[wid], r_v)

    def _chunk(j, _):
        cl = pltpu.async_copy(z_hbm.at[l_v.at[j]], bufl, seml)
        cr = pltpu.async_copy(z_hbm.at[r_v.at[j]], bufr, semr)
        cl.wait()
        cr.wait()

        def _pair(e, __):
            acc = _Z16F()
            for f in range(8):
                acc = acc + bufl[e, pl.ds(f * 16, 16)] * bufr[e, pl.ds(f * 16, 16)]
            pacc[e, pl.ds(0, 16)] = acc
            return 0

        lax.fori_loop(0, 128, _pair, 0)

        for v in range(8):
            idxp = lanes + v * 16
            tot = _Z16F()
            for l in range(16):
                tot = tot + plsc.load_gather(
                    pacc, [idxp, jnp.full((16,), l, jnp.int32)])
            out_v[j, pl.ds(v * 16, 16)] = tot
        return 0

    lax.fori_loop(0, 5, _chunk, 0)
    pltpu.sync_copy(out_v, out_hbm.at[wid])


_dec_call = pl.kernel(
    _dec_body,
    out_type=jax.ShapeDtypeStruct((32, 5, 128), jnp.float32),
    mesh=_MESH,
    compiler_params=_SC_PARAMS,
    scratch_types=[
        pltpu.VMEM((5, 128), jnp.int32),
        pltpu.VMEM((5, 128), jnp.int32),
        pltpu.VMEM((128, 128), jnp.float32),
        pltpu.VMEM((128, 128), jnp.float32),
        pltpu.VMEM((128, 16), jnp.float32),
        pltpu.VMEM((5, 128), jnp.float32),
        pltpu.SemaphoreType.DMA,
        pltpu.SemaphoreType.DMA,
    ],
)


_RB = 256
_NBLK = NP // _RB


def _mm_scale_body(x_ref, w_ref, dis_ref, o_ref):
    g = jnp.dot(x_ref[...], w_ref[...],
                preferred_element_type=jnp.float32) * dis_ref[...]
    o_ref[0] = g[:, :HF]
    o_ref[1] = g[:, HF:]


def _tc_mm_scale(x, w, dis_col):
    return pl.pallas_call(
        _mm_scale_body,
        grid=(_NBLK,),
        in_specs=[
            pl.BlockSpec((_RB, 128), lambda i: (i, 0)),
            pl.BlockSpec((128, 128), lambda i: (0, 0)),
            pl.BlockSpec((_RB, 1), lambda i: (i, 0)),
        ],
        out_specs=pl.BlockSpec((2, _RB, HF), lambda i: (0, i, 0)),
        out_shape=jax.ShapeDtypeStruct((2, NP, HF), jnp.float32),
    )(x, w, dis_col)


def _layer2_body(a_ref, g_ref, dis_ref, b_ref, w_ref, o_ref):
    pre = jnp.concatenate(
        [a_ref[0] + g_ref[0], a_ref[1] + g_ref[1]], axis=-1)
    h = jnp.maximum(dis_ref[...] * pre + b_ref[...], 0.0)
    g2 = jnp.dot(h, w_ref[...],
                 preferred_element_type=jnp.float32) * dis_ref[...]
    o_ref[0] = g2[:, :HF]
    o_ref[1] = g2[:, HF:]


def _tc_layer2(acc, g1, dis_col, b1, w2):
    return pl.pallas_call(
        _layer2_body,
        grid=(_NBLK,),
        in_specs=[
            pl.BlockSpec((2, _RB, HF), lambda i: (0, i, 0)),
            pl.BlockSpec((2, _RB, HF), lambda i: (0, i, 0)),
            pl.BlockSpec((_RB, 1), lambda i: (i, 0)),
            pl.BlockSpec((1, 128), lambda i: (0, 0)),
            pl.BlockSpec((128, 128), lambda i: (0, 0)),
        ],
        out_specs=pl.BlockSpec((2, _RB, HF), lambda i: (0, i, 0)),
        out_shape=jax.ShapeDtypeStruct((2, NP, HF), jnp.float32),
    )(acc, g1, dis_col, b1, w2)


def _final_body(a_ref, g_ref, dis_ref, b_ref, o_ref):
    pre = jnp.concatenate(
        [a_ref[0] + g_ref[0], a_ref[1] + g_ref[1]], axis=-1)
    o_ref[...] = dis_ref[...] * pre + b_ref[...]


def _tc_final(acc, g2, dis_col, b2):
    return pl.pallas_call(
        _final_body,
        grid=(_NBLK,),
        in_specs=[
            pl.BlockSpec((2, _RB, HF), lambda i: (0, i, 0)),
            pl.BlockSpec((2, _RB, HF), lambda i: (0, i, 0)),
            pl.BlockSpec((_RB, 1), lambda i: (i, 0)),
            pl.BlockSpec((1, 128), lambda i: (0, 0)),
        ],
        out_specs=pl.BlockSpec((_RB, 128), lambda i: (i, 0)),
        out_shape=jax.ShapeDtypeStruct((NP, 128), jnp.float32),
    )(acc, g2, dis_col, b2)


def _pad_to(a, total, fill):
    return jnp.concatenate([a, jnp.full((total - a.shape[0],), fill, a.dtype)])


def kernel(x, edge_index, edge_weight, edge_label_index, W1, b1, W2, b2):
    row16 = _pad_to(edge_index[0], EP, 0).reshape(16, CPS, 128)
    col16 = _pad_to(edge_index[1], EP, 0).reshape(16, CPS, 128)
    ew16 = _pad_to(edge_weight, EP, 0.0).reshape(16, CPS, 128)
    x_p = jnp.pad(x, ((0, NP - N), (0, 0)))

    dis2d = _dis_call(col16, ew16)
    dis_col = dis2d.reshape(NP, 1)

    g1 = _tc_mm_scale(x_p, W1, dis_col)
    acc1 = _msg_call(g1, row16, col16, ew16)
    g2 = _tc_layer2(acc1, g1, dis_col, b1.reshape(1, 128), W2)
    acc2 = _msg_call(g2, row16, col16, ew16)
    z = _tc_final(acc2, g2, dis_col, b2.reshape(1, 128))

    lp = _pad_to(edge_label_index[0], ELP, 0).reshape(32, 5, 128)
    rp = _pad_to(edge_label_index[1], ELP, 0).reshape(32, 5, 128)
    logits = _dec_call(z, lp, rp)
    return logits.reshape(-1)[:EL]


# speedup vs baseline: 1.5716x; 1.0062x over previous
import functools

import jax
import jax.numpy as jnp
from jax import lax
from jax.experimental import pallas as pl
from jax.experimental.pallas import tpu as pltpu
from jax.experimental.pallas import tpu_sc as plsc

N = 10000
E = 320000
EL = 20000

NP = 10240
EC = 2560
EP = EC * 128
CPS = EC // 16
ELC = 160
ELP = ELC * 128
HF = 64

_MESH = plsc.VectorSubcoreMesh(core_axis_name="c", subcore_axis_name="s")
_SC_PARAMS = pltpu.CompilerParams(needs_layout_passes=False,
                                  use_tc_tiling_on_sc=False)

_Z16F = functools.partial(jnp.zeros, (16,), jnp.float32)


def _rsqrt16(x):
    i = plsc.bitcast(x, jnp.int32)
    i = jnp.int32(0x5F3759DF) - (i >> 1)
    y = plsc.bitcast(i, jnp.float32)
    for _ in range(3):
        y = y * (1.5 - 0.5 * x * y * y)
    return y


def _dis_body(col_hbm, ew_hbm, dis_hbm, col_v, ew_v, deg_v, st_v, iota_v, deg_sh):
    s = lax.axis_index("s")
    c = lax.axis_index("c")
    zero16 = _Z16F()
    lanes = lax.iota(jnp.int32, 16)

    pltpu.sync_copy(col_hbm.at[s], col_v)
    pltpu.sync_copy(ew_hbm.at[s], ew_v)

    for k in range(5):
        iota_v[pl.ds(k * 16, 16)] = lanes + k * 16

    @pl.when(s < 10)
    def _():
        for r in range(8):
            for f in range(8):
                st_v[r, pl.ds(f * 16, 16)] = zero16
        pltpu.sync_copy(st_v, deg_sh.at[pl.ds(s * 8, 8)])

    def _zb(r, _):
        for f in range(8):
            deg_v[r, pl.ds(f * 16, 16)] = zero16
        return 0

    lax.fori_loop(0, 80, _zb, 0)

    def _eb(j, _):
        for v in range(8):
            idx = col_v[j, pl.ds(v * 16, 16)]
            w = ew_v[j, pl.ds(v * 16, 16)]
            plsc.addupdate_scatter(deg_v, [idx >> 7, idx & 127], w)
        return 0

    lax.fori_loop(0, CPS, _eb, 0)

    plsc.subcore_barrier()
    pltpu.sync_copy(deg_v, deg_sh.at[iota_v], add=True)
    plsc.subcore_barrier()

    @pl.when(jnp.logical_and(s < 10, c == 0))
    def _():
        pltpu.sync_copy(deg_sh.at[pl.ds(s * 8, 8)], st_v)
        for r in range(8):
            for f in range(8):
                d = st_v[r, pl.ds(f * 16, 16)] + 1.0
                st_v[r, pl.ds(f * 16, 16)] = _rsqrt16(d)
        pltpu.sync_copy(st_v, dis_hbm.at[pl.ds(s * 8, 8)])


_dis_call = pl.kernel(
    _dis_body,
    out_type=jax.ShapeDtypeStruct((80, 128), jnp.float32),
    mesh=_MESH,
    compiler_params=_SC_PARAMS,
    scratch_types=[
        pltpu.VMEM((CPS, 128), jnp.int32),
        pltpu.VMEM((CPS, 128), jnp.float32),
        pltpu.VMEM((80, 128), jnp.float32),
        pltpu.VMEM((8, 128), jnp.float32),
        pltpu.VMEM((80,), jnp.int32),
        pltpu.VMEM_SHARED((80, 128), jnp.float32),
    ],
)


def _msg_body(g_hbm, row_hbm, col_hbm, ew_hbm, acc_hbm,
              row_v, col_v, ew_v, buf, z_v, acc_sh, gsem, csem, esem):
    s = lax.axis_index("s")
    c = lax.axis_index("c")
    zero16 = _Z16F()

    pltpu.sync_copy(row_hbm.at[s], row_v)
    ccol = pltpu.async_copy(col_hbm.at[s], col_v, csem)
    cew = pltpu.async_copy(ew_hbm.at[s], ew_v, esem)

    pltpu.async_copy(g_hbm.at[c].at[row_v.at[0]], buf.at[0], gsem.at[0])

    def _zb(r, _):
        for f in range(4):
            z_v[r, pl.ds(f * 16, 16)] = zero16
        return 0

    lax.fori_loop(0, 128, _zb, 0)
    for k in range(5):
        pltpu.sync_copy(z_v, acc_sh.at[pl.ds(s * 640 + k * 128, 128)])
    ccol.wait()
    cew.wait()
    plsc.subcore_barrier()

    def _chunk(j, _):
        b = j & 1
        pltpu.make_async_copy(
            acc_hbm.at[0].at[pl.ds(0, 128)], buf.at[b], gsem.at[b]).wait()

        @pl.when(j < CPS - 1)
        def _():
            pltpu.async_copy(g_hbm.at[c].at[row_v.at[j + 1]],
                             buf.at[1 - b], gsem.at[1 - b])

        row_w = ew_v.at[j]

        def _e8(i, __):
            for u in range(8):
                e = i * 8 + u
                w = plsc.load_gather(row_w, [jnp.zeros((16,), jnp.int32) + e])
                for f in range(4):
                    buf[b, e, pl.ds(f * 16, 16)] = \
                        buf[b, e, pl.ds(f * 16, 16)] * w
            return 0

        lax.fori_loop(0, 16, _e8, 0)
        pltpu.sync_copy(buf.at[b], acc_sh.at[col_v.at[j]], add=True)
        return 0

    lax.fori_loop(0, CPS, _chunk, 0)
    plsc.subcore_barrier()
    pltpu.sync_copy(acc_sh.at[pl.ds(s * 640, 640)],
                    acc_hbm.at[c].at[pl.ds(s * 640, 640)])


_msg_call = pl.kernel(
    _msg_body,
    out_type=jax.ShapeDtypeStruct((2, NP, HF), jnp.float32),
    mesh=_MESH,
    compiler_params=_SC_PARAMS,
    scratch_types=[
        pltpu.VMEM((CPS, 128), jnp.int32),
        pltpu.VMEM((CPS, 128), jnp.int32),
        pltpu.VMEM((CPS, 128), jnp.float32),
        pltpu.VMEM((2, 128, HF), jnp.float32),
        pltpu.VMEM((128, HF), jnp.float32),
        pltpu.VMEM_SHARED((NP, HF), jnp.float32),
        pltpu.SemaphoreType.DMA((2,)),
        pltpu.SemaphoreType.DMA,
        pltpu.SemaphoreType.DMA,
    ],
)


def _dec_body(z_hbm, l_hbm, r_hbm, out_hbm,
              l_v, r_v, bufl, bufr, pacc, out_v, seml, semr):
    s = lax.axis_index("s")
    c = lax.axis_index("c")
    wid = c * 16 + s
    lanes = lax.iota(jnp.int32, 16)

    pltpu.sync_copy(l_hbm.at[wid], l_v)
    pltpu.sync_copy(r_hbm.at[wid], r_v)

    def _chunk(j, _):
        cl = pltpu.async_copy(z_hbm.at[l_v.at[j]], bufl, seml)
        cr = pltpu.async_copy(z_hbm.at[r_v.at[j]], bufr, semr)
        cl.wait()
        cr.wait()

        def _pair(e, __):
            acc = _Z16F()
            for f in range(8):
                acc = acc + bufl[e, pl.ds(f * 16, 16)] * bufr[e, pl.ds(f * 16, 16)]
            pacc[e, pl.ds(0, 16)] = acc
            return 0

        lax.fori_loop(0, 128, _pair, 0)

        for v in range(8):
            idxp = lanes + v * 16
            tot = _Z16F()
            for l in range(16):
                tot = tot + plsc.load_gather(
                    pacc, [idxp, jnp.full((16,), l, jnp.int32)])
            out_v[j, pl.ds(v * 16, 16)] = tot
        return 0

    lax.fori_loop(0, 5, _chunk, 0)
    pltpu.sync_copy(out_v, out_hbm.at[wid])


_dec_call = pl.kernel(
    _dec_body,
    out_type=jax.ShapeDtypeStruct((32, 5, 128), jnp.float32),
    mesh=_MESH,
    compiler_params=_SC_PARAMS,
    scratch_types=[
        pltpu.VMEM((5, 128), jnp.int32),
        pltpu.VMEM((5, 128), jnp.int32),
        pltpu.VMEM((128, 128), jnp.float32),
        pltpu.VMEM((128, 128), jnp.float32),
        pltpu.VMEM((128, 16), jnp.float32),
        pltpu.VMEM((5, 128), jnp.float32),
        pltpu.SemaphoreType.DMA,
        pltpu.SemaphoreType.DMA,
    ],
)


_RB = 256
_NBLK = NP // _RB


def _mm_scale_body(x_ref, w_ref, dis_ref, o_ref):
    g = jnp.dot(x_ref[...], w_ref[...],
                preferred_element_type=jnp.float32) * dis_ref[...]
    o_ref[0] = g[:, :HF]
    o_ref[1] = g[:, HF:]


def _tc_mm_scale(x, w, dis_col):
    return pl.pallas_call(
        _mm_scale_body,
        grid=(_NBLK,),
        in_specs=[
            pl.BlockSpec((_RB, 128), lambda i: (i, 0)),
            pl.BlockSpec((128, 128), lambda i: (0, 0)),
            pl.BlockSpec((_RB, 1), lambda i: (i, 0)),
        ],
        out_specs=pl.BlockSpec((2, _RB, HF), lambda i: (0, i, 0)),
        out_shape=jax.ShapeDtypeStruct((2, NP, HF), jnp.float32),
    )(x, w, dis_col)


def _layer2_body(a_ref, g_ref, dis_ref, b_ref, w_ref, o_ref):
    pre = jnp.concatenate(
        [a_ref[0] + g_ref[0], a_ref[1] + g_ref[1]], axis=-1)
    h = jnp.maximum(dis_ref[...] * pre + b_ref[...], 0.0)
    g2 = jnp.dot(h, w_ref[...],
                 preferred_element_type=jnp.float32) * dis_ref[...]
    o_ref[0] = g2[:, :HF]
    o_ref[1] = g2[:, HF:]


def _tc_layer2(acc, g1, dis_col, b1, w2):
    return pl.pallas_call(
        _layer2_body,
        grid=(_NBLK,),
        in_specs=[
            pl.BlockSpec((2, _RB, HF), lambda i: (0, i, 0)),
            pl.BlockSpec((2, _RB, HF), lambda i: (0, i, 0)),
            pl.BlockSpec((_RB, 1), lambda i: (i, 0)),
            pl.BlockSpec((1, 128), lambda i: (0, 0)),
            pl.BlockSpec((128, 128), lambda i: (0, 0)),
        ],
        out_specs=pl.BlockSpec((2, _RB, HF), lambda i: (0, i, 0)),
        out_shape=jax.ShapeDtypeStruct((2, NP, HF), jnp.float32),
    )(acc, g1, dis_col, b1, w2)


def _final_body(a_ref, g_ref, dis_ref, b_ref, o_ref):
    pre = jnp.concatenate(
        [a_ref[0] + g_ref[0], a_ref[1] + g_ref[1]], axis=-1)
    o_ref[...] = dis_ref[...] * pre + b_ref[...]


def _tc_final(acc, g2, dis_col, b2):
    return pl.pallas_call(
        _final_body,
        grid=(_NBLK,),
        in_specs=[
            pl.BlockSpec((2, _RB, HF), lambda i: (0, i, 0)),
            pl.BlockSpec((2, _RB, HF), lambda i: (0, i, 0)),
            pl.BlockSpec((_RB, 1), lambda i: (i, 0)),
            pl.BlockSpec((1, 128), lambda i: (0, 0)),
        ],
        out_specs=pl.BlockSpec((_RB, 128), lambda i: (i, 0)),
        out_shape=jax.ShapeDtypeStruct((NP, 128), jnp.float32),
    )(acc, g2, dis_col, b2)


def _pad_to(a, total, fill):
    return jnp.concatenate([a, jnp.full((total - a.shape[0],), fill, a.dtype)])


def kernel(x, edge_index, edge_weight, edge_label_index, W1, b1, W2, b2):
    row16 = _pad_to(edge_index[0], EP, 0).reshape(16, CPS, 128)
    col16 = _pad_to(edge_index[1], EP, 0).reshape(16, CPS, 128)
    ew16 = _pad_to(edge_weight, EP, 0.0).reshape(16, CPS, 128)
    x_p = jnp.pad(x, ((0, NP - N), (0, 0)))

    dis2d = _dis_call(col16, ew16)
    dis_col = dis2d.reshape(NP, 1)

    g1 = _tc_mm_scale(x_p, W1, dis_col)
    acc1 = _msg_call(g1, row16, col16, ew16)
    g2 = _tc_layer2(acc1, g1, dis_col, b1.reshape(1, 128), W2)
    acc2 = _msg_call(g2, row16, col16, ew16)
    z = _tc_final(acc2, g2, dis_col, b2.reshape(1, 128))

    lp = _pad_to(edge_label_index[0], ELP, 0).reshape(32, 5, 128)
    rp = _pad_to(edge_label_index[1], ELP, 0).reshape(32, 5, 128)
    logits = _dec_call(z, lp, rp)
    return logits.reshape(-1)[:EL]
